# baseline (speedup 1.0000x reference)
import functools

import jax
import jax.numpy as jnp
import numpy as np
from jax import lax
from jax.experimental import pallas as pl
from jax.experimental.pallas import tpu as pltpu
from jax.experimental.pallas import tpu_sc as plsc

_MASK_FREQ = 0.15
_TOTAL_TOKENS = 32768
_D_FEAT = 512


def _rotl32(x, r):
    return ((x << np.uint32(r)) | (x >> np.uint32(32 - r))).astype(np.uint32)


def _threefry2x32(k0, k1, x0, x1):
    rotations = ((13, 15, 26, 6), (17, 29, 16, 24))
    ks = (
        np.uint32(k0),
        np.uint32(k1),
        np.uint32(k0) ^ np.uint32(k1) ^ np.uint32(0x1BD11BDA),
    )
    x0 = (x0 + ks[0]).astype(np.uint32)
    x1 = (x1 + ks[1]).astype(np.uint32)
    for d in range(5):
        for rot in rotations[d % 2]:
            x0 = (x0 + x1).astype(np.uint32)
            x1 = _rotl32(x1, rot)
            x1 = (x1 ^ x0).astype(np.uint32)
        x0 = (x0 + ks[(d + 1) % 3]).astype(np.uint32)
        x1 = (x1 + ks[(d + 2) % 3] + np.uint32(d + 1)).astype(np.uint32)
    return x0, x1


def _uniform_key42(n):
    lo = np.arange(n, dtype=np.uint32)
    hi = np.zeros(n, np.uint32)
    x0, x1 = _threefry2x32(0, 42, hi, lo)
    bits = (x0 ^ x1).astype(np.uint32)
    return ((bits >> np.uint32(9)) | np.uint32(0x3F800000)).view(np.float32) - np.float32(1.0)


_KEPT = _uniform_key42(_TOTAL_TOKENS) > _MASK_FREQ
_N_KEPT_ROWS = int(_KEPT.sum())
_IDX = np.nonzero(_KEPT)[0].astype(np.int32)

_NW = 32
_NC = 2
_CH = 112
_NCHUNKS = -(-_N_KEPT_ROWS // _CH)
_TAIL = _N_KEPT_ROWS - (_NCHUNKS - 1) * _CH
_ITERS = -(-_NCHUNKS // _NW)

_IDX_PAD = np.full((_NCHUNKS * _CH,), _IDX[-1], np.int32)
_IDX_PAD[:_N_KEPT_ROWS] = _IDX

_IDX_WORKER = np.full((_NW, _ITERS * _CH), _IDX[-1], np.int32)
for _w in range(_NW):
    for _i in range(_ITERS):
        _c = _w + _i * _NW
        if _c < _NCHUNKS:
            _IDX_WORKER[_w, _i * _CH:(_i + 1) * _CH] = _IDX_PAD[_c * _CH:(_c + 1) * _CH]
_IDX_WORKER = _IDX_WORKER.reshape(-1)

_DST_TAIL = np.minimum(
    np.arange((_NCHUNKS - 1) * _CH, _NCHUNKS * _CH), _N_KEPT_ROWS - 1
).astype(np.int32)


@functools.cache
def _build_sc_gather():
    mesh = plsc.VectorSubcoreMesh(core_axis_name="c", subcore_axis_name="s")

    @functools.partial(
        pl.kernel,
        mesh=mesh,
        out_type=jax.ShapeDtypeStruct((_N_KEPT_ROWS, _D_FEAT), jnp.float32),
        scratch_types=[
            pltpu.VMEM((_ITERS * _CH,), jnp.int32),
            pltpu.VMEM((_CH,), jnp.int32),
            pltpu.VMEM((_CH, _D_FEAT), jnp.float32),
            pltpu.VMEM((_CH, _D_FEAT), jnp.float32),
            pltpu.SemaphoreType.DMA,
            pltpu.SemaphoreType.DMA,
            pltpu.SemaphoreType.DMA,
            pltpu.SemaphoreType.DMA,
        ],
    )
    def _sc_gather(feat_hbm, idx_hbm, dst_hbm, out_hbm,
                   idx_v, dst_v, buf0, buf1, g0, g1, s0, s1):
        w = lax.axis_index("s") * _NC + lax.axis_index("c")
        bufs = (buf0, buf1)
        gsems = (g0, g1)
        ssems = (s0, s1)

        pltpu.sync_copy(
            idx_hbm.at[pl.ds(w * (_ITERS * _CH), _ITERS * _CH)], idx_v
        )

        def start_gather(i):
            b = i % 2
            c = w + i * _NW

            @pl.when(c < _NCHUNKS)
            def _():
                pltpu.async_copy(
                    feat_hbm.at[idx_v.at[pl.ds(i * _CH, _CH)]], bufs[b], gsems[b]
                )

        def finish_chunk(i):
            b = i % 2
            c = w + i * _NW

            @pl.when(c < _NCHUNKS)
            def _():
                pltpu.make_async_copy(
                    feat_hbm.at[idx_v.at[pl.ds(i * _CH, _CH)]], bufs[b], gsems[b]
                ).wait()

                @pl.when(c < _NCHUNKS - 1)
                def _full():
                    pltpu.async_copy(
                        bufs[b], out_hbm.at[pl.ds(c * _CH, _CH)], ssems[b]
                    )

                @pl.when(c == _NCHUNKS - 1)
                def _tail():
                    pltpu.sync_copy(dst_hbm, dst_v)
                    pltpu.async_copy(bufs[b], out_hbm.at[dst_v], ssems[b])

        def wait_store(i):
            b = i % 2
            c = w + i * _NW

            @pl.when(c < _NCHUNKS - 1)
            def _():
                pltpu.make_async_copy(
                    bufs[b], out_hbm.at[pl.ds(c * _CH, _CH)], ssems[b]
                ).wait()

            @pl.when(c == _NCHUNKS - 1)
            def _():
                pltpu.make_async_copy(bufs[b], out_hbm.at[dst_v], ssems[b]).wait()

        start_gather(0)
        for i in range(_ITERS):
            if i + 1 < _ITERS:
                if i - 1 >= 0:
                    wait_store(i - 1)
                start_gather(i + 1)
            finish_chunk(i)
        for i in range(max(0, _ITERS - 2), _ITERS):
            wait_store(i)

    return _sc_gather


def kernel(feature):
    return _build_sc_gather()(
        feature, jnp.asarray(_IDX_WORKER), jnp.asarray(_DST_TAIL)
    )


# speedup vs baseline: 1.0453x; 1.0453x over previous
import functools

import jax
import jax.numpy as jnp
import numpy as np
from jax import lax
from jax.experimental import pallas as pl
from jax.experimental.pallas import tpu as pltpu
from jax.experimental.pallas import tpu_sc as plsc

_MASK_FREQ = 0.15
_TOTAL_TOKENS = 32768
_D_FEAT = 512


def _rotl32(x, r):
    return ((x << np.uint32(r)) | (x >> np.uint32(32 - r))).astype(np.uint32)


def _threefry2x32(k0, k1, x0, x1):
    rotations = ((13, 15, 26, 6), (17, 29, 16, 24))
    ks = (
        np.uint32(k0),
        np.uint32(k1),
        np.uint32(k0) ^ np.uint32(k1) ^ np.uint32(0x1BD11BDA),
    )
    x0 = (x0 + ks[0]).astype(np.uint32)
    x1 = (x1 + ks[1]).astype(np.uint32)
    for d in range(5):
        for rot in rotations[d % 2]:
            x0 = (x0 + x1).astype(np.uint32)
            x1 = _rotl32(x1, rot)
            x1 = (x1 ^ x0).astype(np.uint32)
        x0 = (x0 + ks[(d + 1) % 3]).astype(np.uint32)
        x1 = (x1 + ks[(d + 2) % 3] + np.uint32(d + 1)).astype(np.uint32)
    return x0, x1


def _uniform_key42(n):
    lo = np.arange(n, dtype=np.uint32)
    hi = np.zeros(n, np.uint32)
    x0, x1 = _threefry2x32(0, 42, hi, lo)
    bits = (x0 ^ x1).astype(np.uint32)
    return ((bits >> np.uint32(9)) | np.uint32(0x3F800000)).view(np.float32) - np.float32(1.0)


_KEPT = _uniform_key42(_TOTAL_TOKENS) > _MASK_FREQ
_N_KEPT_ROWS = int(_KEPT.sum())
_IDX = np.nonzero(_KEPT)[0].astype(np.int32)

_NW = 32
_NC = 2
_CH = 64
_BASE = 872
_LAST = _N_KEPT_ROWS - (_NW - 1) * _BASE
_SLOTS = -(-_BASE // _CH)
_REM = _BASE - (_SLOTS - 1) * _CH
_LAST_FULL = _LAST // _CH
_LAST_TAIL = _LAST - _LAST_FULL * _CH

_IDX_HOST = np.full((_NW * _BASE,), _IDX[-1], np.int32)
_IDX_HOST[:_N_KEPT_ROWS] = _IDX

_DST_TAIL = np.minimum(
    np.arange((_NW - 1) * _BASE + _LAST_FULL * _CH,
              (_NW - 1) * _BASE + _LAST_FULL * _CH + _CH),
    _N_KEPT_ROWS - 1,
).astype(np.int32)


@functools.cache
def _build_sc_gather():
    mesh = plsc.VectorSubcoreMesh(core_axis_name="c", subcore_axis_name="s")

    @functools.partial(
        pl.kernel,
        mesh=mesh,
        out_type=jax.ShapeDtypeStruct((_N_KEPT_ROWS, _D_FEAT), jnp.float32),
        scratch_types=[
            pltpu.VMEM((_BASE,), jnp.int32),
            pltpu.VMEM((_CH,), jnp.int32),
            pltpu.VMEM((_CH, _D_FEAT), jnp.float32),
            pltpu.VMEM((_CH, _D_FEAT), jnp.float32),
            pltpu.SemaphoreType.DMA,
            pltpu.SemaphoreType.DMA,
            pltpu.SemaphoreType.DMA,
            pltpu.SemaphoreType.DMA,
        ],
    )
    def _sc_gather(feat_hbm, idx_hbm, dst_hbm, out_hbm,
                   idx_v, dst_v, buf0, buf1, g0, g1, s0, s1):
        w = lax.axis_index("s") * _NC + lax.axis_index("c")
        base = w * _BASE
        bufs = (buf0, buf1)
        gsems = (g0, g1)
        ssems = (s0, s1)
        last_w = _NW - 1

        pltpu.sync_copy(idx_hbm.at[pl.ds(base, _BASE)], idx_v)
        pltpu.sync_copy(dst_hbm, dst_v)

        def n_gather(s):
            return _CH if s < _SLOTS - 1 else _REM

        def start_gather(s):
            b = s % 2
            n = n_gather(s)

            def issue():
                pltpu.async_copy(
                    feat_hbm.at[idx_v.at[pl.ds(s * _CH, n)]],
                    bufs[b] if n == _CH else bufs[b].at[pl.ds(0, n)],
                    gsems[b],
                )

            if s < _SLOTS - 1:
                issue()
            else:
                pl.when(w < last_w)(issue)

        def wait_gather(s):
            b = s % 2
            n = n_gather(s)
            pltpu.make_async_copy(
                feat_hbm.at[idx_v.at[pl.ds(s * _CH, n)]],
                bufs[b] if n == _CH else bufs[b].at[pl.ds(0, n)],
                gsems[b],
            ).wait()

        def start_store(s):
            b = s % 2
            if s < _LAST_FULL:
                pltpu.async_copy(
                    bufs[b], out_hbm.at[pl.ds(base + s * _CH, _CH)], ssems[b]
                )
            elif s == _LAST_FULL:
                @pl.when(w < last_w)
                def _():
                    pltpu.async_copy(
                        bufs[b], out_hbm.at[pl.ds(base + s * _CH, _CH)], ssems[b]
                    )

                @pl.when(w == last_w)
                def _():
                    pltpu.async_copy(bufs[b], out_hbm.at[dst_v], ssems[b])
            else:
                @pl.when(w < last_w)
                def _():
                    pltpu.async_copy(
                        bufs[b].at[pl.ds(0, _REM)],
                        out_hbm.at[pl.ds(base + s * _CH, _REM)],
                        ssems[b],
                    )

        def wait_store(s):
            b = s % 2
            if s < _LAST_FULL:
                pltpu.make_async_copy(
                    bufs[b], out_hbm.at[pl.ds(base + s * _CH, _CH)], ssems[b]
                ).wait()
            elif s == _LAST_FULL:
                @pl.when(w < last_w)
                def _():
                    pltpu.make_async_copy(
                        bufs[b], out_hbm.at[pl.ds(base + s * _CH, _CH)], ssems[b]
                    ).wait()

                @pl.when(w == last_w)
                def _():
                    pltpu.make_async_copy(
                        bufs[b], out_hbm.at[dst_v], ssems[b]
                    ).wait()
            else:
                @pl.when(w < last_w)
                def _():
                    pltpu.make_async_copy(
                        bufs[b].at[pl.ds(0, _REM)],
                        out_hbm.at[pl.ds(base + s * _CH, _REM)],
                        ssems[b],
                    ).wait()

        def finish_chunk(s):
            def run():
                wait_gather(s)
                start_store(s)

            if s < _SLOTS - 1:
                run()
            else:
                pl.when(w < last_w)(run)

        start_gather(0)
        for s in range(_SLOTS):
            if s + 1 < _SLOTS:
                if s - 1 >= 0:
                    wait_store(s - 1)
                start_gather(s + 1)
            finish_chunk(s)
        for s in range(_SLOTS - 2, _SLOTS):
            wait_store(s)

    return _sc_gather


def kernel(feature):
    return _build_sc_gather()(
        feature, jnp.asarray(_IDX_HOST), jnp.asarray(_DST_TAIL)
    )


# speedup vs baseline: 1.0779x; 1.0312x over previous
import functools

import jax
import jax.numpy as jnp
import numpy as np
from jax import lax
from jax.experimental import pallas as pl
from jax.experimental.pallas import tpu as pltpu
from jax.experimental.pallas import tpu_sc as plsc

_MASK_FREQ = 0.15
_TOTAL_TOKENS = 32768
_D_FEAT = 512


def _rotl32(x, r):
    return ((x << np.uint32(r)) | (x >> np.uint32(32 - r))).astype(np.uint32)


def _threefry2x32(k0, k1, x0, x1):
    rotations = ((13, 15, 26, 6), (17, 29, 16, 24))
    ks = (
        np.uint32(k0),
        np.uint32(k1),
        np.uint32(k0) ^ np.uint32(k1) ^ np.uint32(0x1BD11BDA),
    )
    x0 = (x0 + ks[0]).astype(np.uint32)
    x1 = (x1 + ks[1]).astype(np.uint32)
    for d in range(5):
        for rot in rotations[d % 2]:
            x0 = (x0 + x1).astype(np.uint32)
            x1 = _rotl32(x1, rot)
            x1 = (x1 ^ x0).astype(np.uint32)
        x0 = (x0 + ks[(d + 1) % 3]).astype(np.uint32)
        x1 = (x1 + ks[(d + 2) % 3] + np.uint32(d + 1)).astype(np.uint32)
    return x0, x1


def _uniform_key42(n):
    lo = np.arange(n, dtype=np.uint32)
    hi = np.zeros(n, np.uint32)
    x0, x1 = _threefry2x32(0, 42, hi, lo)
    bits = (x0 ^ x1).astype(np.uint32)
    return ((bits >> np.uint32(9)) | np.uint32(0x3F800000)).view(np.float32) - np.float32(1.0)


_KEPT = _uniform_key42(_TOTAL_TOKENS) > _MASK_FREQ
_N_KEPT_ROWS = int(_KEPT.sum())
_IDX = np.nonzero(_KEPT)[0].astype(np.int32)

_NW = 32
_NC = 2
_CH = 64
_NCHUNKS = -(-_N_KEPT_ROWS // _CH)
_TAIL = _N_KEPT_ROWS - (_NCHUNKS - 1) * _CH
_SLOTS = -(-_NCHUNKS // _NW)
_TAIL_W = (_NCHUNKS - 1) % _NW

_IDX_PAD = np.full((_NW * _SLOTS * _CH,), _IDX[-1], np.int32)
_IDX_PAD[:_N_KEPT_ROWS] = _IDX
_IDX_WORKER = np.full((_NW, _SLOTS * _CH), _IDX[-1], np.int32)
for _w in range(_NW):
    for _s in range(_SLOTS):
        _c = _w + _s * _NW
        if _c < _NCHUNKS:
            _IDX_WORKER[_w, _s * _CH:(_s + 1) * _CH] = _IDX_PAD[_c * _CH:(_c + 1) * _CH]
_IDX_WORKER = _IDX_WORKER.reshape(-1)

_DST_TAIL = np.minimum(
    np.arange((_NCHUNKS - 1) * _CH, _NCHUNKS * _CH), _N_KEPT_ROWS - 1
).astype(np.int32)


@functools.cache
def _build_sc_gather():
    mesh = plsc.VectorSubcoreMesh(core_axis_name="c", subcore_axis_name="s")

    @functools.partial(
        pl.kernel,
        mesh=mesh,
        out_type=jax.ShapeDtypeStruct((_N_KEPT_ROWS, _D_FEAT), jnp.float32),
        scratch_types=[
            pltpu.VMEM((_SLOTS * _CH,), jnp.int32),
            pltpu.VMEM((_CH,), jnp.int32),
            pltpu.VMEM((_CH, _D_FEAT), jnp.float32),
            pltpu.VMEM((_CH, _D_FEAT), jnp.float32),
            pltpu.SemaphoreType.DMA,
            pltpu.SemaphoreType.DMA,
            pltpu.SemaphoreType.DMA,
            pltpu.SemaphoreType.DMA,
        ],
    )
    def _sc_gather(feat_hbm, idx_hbm, dst_hbm, out_hbm,
                   idx_v, dst_v, buf0, buf1, g0, g1, s0, s1):
        w = lax.axis_index("s") * _NC + lax.axis_index("c")
        bufs = (buf0, buf1)
        gsems = (g0, g1)
        ssems = (s0, s1)

        pltpu.sync_copy(
            idx_hbm.at[pl.ds(w * (_SLOTS * _CH), _SLOTS * _CH)], idx_v
        )
        pltpu.sync_copy(dst_hbm, dst_v)

        def start_gather(s):
            b = s % 2

            def issue():
                pltpu.async_copy(
                    feat_hbm.at[idx_v.at[pl.ds(s * _CH, _CH)]], bufs[b], gsems[b]
                )

            if s < _SLOTS - 1:
                issue()
            else:
                pl.when(w <= _TAIL_W)(issue)

        def wait_gather(s):
            b = s % 2
            pltpu.make_async_copy(
                feat_hbm.at[idx_v.at[pl.ds(s * _CH, _CH)]], bufs[b], gsems[b]
            ).wait()

        def start_store(s):
            b = s % 2
            if s < _SLOTS - 1:
                pltpu.async_copy(
                    bufs[b], out_hbm.at[pl.ds((w + s * _NW) * _CH, _CH)], ssems[b]
                )
            else:
                @pl.when(w < _TAIL_W)
                def _():
                    pltpu.async_copy(
                        bufs[b], out_hbm.at[pl.ds((w + s * _NW) * _CH, _CH)],
                        ssems[b],
                    )

                @pl.when(w == _TAIL_W)
                def _():
                    pltpu.async_copy(bufs[b], out_hbm.at[dst_v], ssems[b])

        def wait_store(s):
            b = s % 2
            if s < _SLOTS - 1:
                pltpu.make_async_copy(
                    bufs[b], out_hbm.at[pl.ds((w + s * _NW) * _CH, _CH)], ssems[b]
                ).wait()
            else:
                @pl.when(w < _TAIL_W)
                def _():
                    pltpu.make_async_copy(
                        bufs[b], out_hbm.at[pl.ds((w + s * _NW) * _CH, _CH)],
                        ssems[b],
                    ).wait()

                @pl.when(w == _TAIL_W)
                def _():
                    pltpu.make_async_copy(
                        bufs[b], out_hbm.at[dst_v], ssems[b]
                    ).wait()

        def finish_chunk(s):
            def run():
                wait_gather(s)
                start_store(s)

            if s < _SLOTS - 1:
                run()
            else:
                pl.when(w <= _TAIL_W)(run)

        start_gather(0)
        for s in range(_SLOTS):
            if s + 1 < _SLOTS:
                if s - 1 >= 0:
                    wait_store(s - 1)
                start_gather(s + 1)
            finish_chunk(s)
        for s in range(_SLOTS - 2, _SLOTS):
            wait_store(s)

    return _sc_gather


def kernel(feature):
    return _build_sc_gather()(
        feature, jnp.asarray(_IDX_WORKER), jnp.asarray(_DST_TAIL)
    )


# speedup vs baseline: 1.0812x; 1.0031x over previous
import functools

import jax
import jax.numpy as jnp
import numpy as np
from jax import lax
from jax.experimental import pallas as pl
from jax.experimental.pallas import tpu as pltpu
from jax.experimental.pallas import tpu_sc as plsc

_MASK_FREQ = 0.15
_TOTAL_TOKENS = 32768
_D_FEAT = 512


def _rotl32(x, r):
    return ((x << np.uint32(r)) | (x >> np.uint32(32 - r))).astype(np.uint32)


def _threefry2x32(k0, k1, x0, x1):
    rotations = ((13, 15, 26, 6), (17, 29, 16, 24))
    ks = (
        np.uint32(k0),
        np.uint32(k1),
        np.uint32(k0) ^ np.uint32(k1) ^ np.uint32(0x1BD11BDA),
    )
    x0 = (x0 + ks[0]).astype(np.uint32)
    x1 = (x1 + ks[1]).astype(np.uint32)
    for d in range(5):
        for rot in rotations[d % 2]:
            x0 = (x0 + x1).astype(np.uint32)
            x1 = _rotl32(x1, rot)
            x1 = (x1 ^ x0).astype(np.uint32)
        x0 = (x0 + ks[(d + 1) % 3]).astype(np.uint32)
        x1 = (x1 + ks[(d + 2) % 3] + np.uint32(d + 1)).astype(np.uint32)
    return x0, x1


def _uniform_key42(n):
    lo = np.arange(n, dtype=np.uint32)
    hi = np.zeros(n, np.uint32)
    x0, x1 = _threefry2x32(0, 42, hi, lo)
    bits = (x0 ^ x1).astype(np.uint32)
    return ((bits >> np.uint32(9)) | np.uint32(0x3F800000)).view(np.float32) - np.float32(1.0)


_KEPT = _uniform_key42(_TOTAL_TOKENS) > _MASK_FREQ
_N_KEPT_ROWS = int(_KEPT.sum())
_IDX = np.nonzero(_KEPT)[0].astype(np.int32)

_NW = 32
_NC = 2
_CH = 64
_NCHUNKS = -(-_N_KEPT_ROWS // _CH)
_TAIL = _N_KEPT_ROWS - (_NCHUNKS - 1) * _CH
_SLOTS = -(-_NCHUNKS // _NW)
_TAIL_W = (_NCHUNKS - 1) % _NW

_IDX_PAD = np.full((_NW * _SLOTS * _CH,), _IDX[-1], np.int32)
_IDX_PAD[:_N_KEPT_ROWS] = _IDX
_IDX_WORKER = np.full((_NW, _SLOTS * _CH), _IDX[-1], np.int32)
for _w in range(_NW):
    for _s in range(_SLOTS):
        _c = _w + _s * _NW
        if _c < _NCHUNKS:
            _IDX_WORKER[_w, _s * _CH:(_s + 1) * _CH] = _IDX_PAD[_c * _CH:(_c + 1) * _CH]
_IDX_WORKER = _IDX_WORKER.reshape(-1)

_DST_TAIL = np.minimum(
    np.arange((_NCHUNKS - 1) * _CH, _NCHUNKS * _CH), _N_KEPT_ROWS - 1
).astype(np.int32)


@functools.cache
def _build_sc_gather():
    mesh = plsc.VectorSubcoreMesh(core_axis_name="c", subcore_axis_name="s")

    @functools.partial(
        pl.kernel,
        mesh=mesh,
        out_type=jax.ShapeDtypeStruct((_N_KEPT_ROWS, _D_FEAT), jnp.float32),
        scratch_types=[
            pltpu.VMEM((_SLOTS * _CH,), jnp.int32),
            pltpu.VMEM((_CH,), jnp.int32),
            pltpu.VMEM((_CH, _D_FEAT), jnp.float32),
            pltpu.VMEM((_CH, _D_FEAT), jnp.float32),
            pltpu.VMEM((_CH, _D_FEAT), jnp.float32),
            pltpu.SemaphoreType.DMA,
            pltpu.SemaphoreType.DMA,
            pltpu.SemaphoreType.DMA,
            pltpu.SemaphoreType.DMA,
            pltpu.SemaphoreType.DMA,
            pltpu.SemaphoreType.DMA,
        ],
    )
    def _sc_gather(feat_hbm, idx_hbm, dst_hbm, out_hbm,
                   idx_v, dst_v, buf0, buf1, buf2, g0, g1, g2, s0, s1, s2):
        w = lax.axis_index("s") * _NC + lax.axis_index("c")
        bufs = (buf0, buf1, buf2)
        gsems = (g0, g1, g2)
        ssems = (s0, s1, s2)

        pltpu.sync_copy(
            idx_hbm.at[pl.ds(w * (_SLOTS * _CH), _SLOTS * _CH)], idx_v
        )
        pltpu.sync_copy(dst_hbm, dst_v)

        def start_gather(s):
            b = s % 3

            def issue():
                pltpu.async_copy(
                    feat_hbm.at[idx_v.at[pl.ds(s * _CH, _CH)]], bufs[b], gsems[b]
                )

            if s < _SLOTS - 1:
                issue()
            else:
                pl.when(w <= _TAIL_W)(issue)

        def wait_gather(s):
            b = s % 3
            pltpu.make_async_copy(
                feat_hbm.at[idx_v.at[pl.ds(s * _CH, _CH)]], bufs[b], gsems[b]
            ).wait()

        def start_store(s):
            b = s % 3
            if s < _SLOTS - 1:
                pltpu.async_copy(
                    bufs[b], out_hbm.at[pl.ds((w + s * _NW) * _CH, _CH)], ssems[b]
                )
            else:
                @pl.when(w < _TAIL_W)
                def _():
                    pltpu.async_copy(
                        bufs[b], out_hbm.at[pl.ds((w + s * _NW) * _CH, _CH)],
                        ssems[b],
                    )

                @pl.when(w == _TAIL_W)
                def _():
                    pltpu.async_copy(bufs[b], out_hbm.at[dst_v], ssems[b])

        def wait_store(s):
            b = s % 3
            if s < _SLOTS - 1:
                pltpu.make_async_copy(
                    bufs[b], out_hbm.at[pl.ds((w + s * _NW) * _CH, _CH)], ssems[b]
                ).wait()
            else:
                @pl.when(w < _TAIL_W)
                def _():
                    pltpu.make_async_copy(
                        bufs[b], out_hbm.at[pl.ds((w + s * _NW) * _CH, _CH)],
                        ssems[b],
                    ).wait()

                @pl.when(w == _TAIL_W)
                def _():
                    pltpu.make_async_copy(
                        bufs[b], out_hbm.at[dst_v], ssems[b]
                    ).wait()

        def finish_chunk(s):
            def run():
                wait_gather(s)
                start_store(s)

            if s < _SLOTS - 1:
                run()
            else:
                pl.when(w <= _TAIL_W)(run)

        start_gather(0)
        start_gather(1)
        for s in range(_SLOTS):
            if s + 2 < _SLOTS:
                if s - 1 >= 0:
                    wait_store(s - 1)
                start_gather(s + 2)
            finish_chunk(s)
        for s in range(_SLOTS - 3, _SLOTS):
            wait_store(s)

    return _sc_gather


def kernel(feature):
    return _build_sc_gather()(
        feature, jnp.asarray(_IDX_WORKER), jnp.asarray(_DST_TAIL)
    )
